# Initial kernel scaffold; baseline (speedup 1.0000x reference)
import jax
import jax.numpy as jnp
from jax.experimental import pallas as pl


def kernel(x, edge_index, edge_label_index, W1, b1, W2, b2, W3, b3):
    raise NotImplementedError("write your pallas kernel here")


# speedup vs baseline: 1.0140x; 1.0140x over previous
import jax
import jax.numpy as jnp
from jax.experimental import pallas as pl


def _gcn_conv(x, edge_index, W, b, n_nodes):
    x = x @ W
    self_loops = jnp.arange(n_nodes, dtype=edge_index.dtype)
    row = jnp.concatenate([edge_index[0], self_loops])
    col = jnp.concatenate([edge_index[1], self_loops])
    deg = jax.ops.segment_sum(jnp.ones_like(col, dtype=x.dtype), col, num_segments=n_nodes)
    deg_inv_sqrt = jnp.where(deg > 0, deg ** -0.5, 0.0)
    norm = deg_inv_sqrt[row] * deg_inv_sqrt[col]
    msgs = x[row] * norm[:, None]
    out = jax.ops.segment_sum(msgs, col, num_segments=n_nodes)
    return out + b


def _decode_body(a_ref, b_ref, o_ref):
    o_ref[pl.program_id(0), :] = jnp.sum(a_ref[...] * b_ref[...], axis=-1)


def kernel(x, edge_index, edge_label_index, W1, b1, W2, b2, W3, b3):
    n = x.shape[0]
    z = _gcn_conv(x, edge_index, W1, b1, n)
    z = jax.nn.relu(z)
    z = _gcn_conv(z, edge_index, W2, b2, n)
    z = jax.nn.relu(z)
    z = _gcn_conv(z, edge_index, W3, b3, n)
    a = z[edge_label_index[0]]
    b = z[edge_label_index[1]]
    E = a.shape[0]
    BE = 20000
    out = pl.pallas_call(
        _decode_body,
        grid=(E // BE,),
        in_specs=[pl.BlockSpec((BE, a.shape[1]), lambda i: (i, 0)),
                  pl.BlockSpec((BE, a.shape[1]), lambda i: (i, 0))],
        out_specs=pl.BlockSpec((E // BE, BE), lambda i: (0, 0)),
        out_shape=jax.ShapeDtypeStruct((E // BE, BE), jnp.float32),
    )(a, b)
    out = out.reshape(E)
    return out


# speedup vs baseline: 9.3352x; 9.2062x over previous
import functools

import jax
import jax.numpy as jnp
from jax import lax
from jax.experimental import pallas as pl
from jax.experimental.pallas import tpu as pltpu
from jax.experimental.pallas import tpu_sc as plsc

N = 10000
E = 320000
EL = 200000
NC, NS = 2, 16
NW = NC * NS
CHUNK = 125
WB_TILES = 10
WB_ROWS = N // WB_TILES
E_PER_TILE = E // NW
E_CHUNKS = E_PER_TILE // CHUNK
ELP = 204800
EL_PER_TILE = ELP // NW
CHUNK_D = 128
EL_CHUNKS = EL_PER_TILE // CHUNK_D
GROUPS_D = CHUNK_D // 16

_MESH = plsc.VectorSubcoreMesh(core_axis_name="c", subcore_axis_name="s")


def _wid():
    return lax.axis_index("s") * NC + lax.axis_index("c")


DEG_W = 128
_DEG_KW = dict(
    out_type=jax.ShapeDtypeStruct((NC, N, DEG_W), jnp.float32),
    mesh=_MESH,
    scratch_types=[
        pltpu.VMEM((E_CHUNKS, CHUNK), jnp.int32),
        pltpu.VMEM((CHUNK, DEG_W), jnp.float32),
        pltpu.VMEM_SHARED((N, DEG_W), jnp.float32),
    ],
)


def _sc_degree_body(col_hbm, ones_hbm, zeros_hbm, out_hbm, col_v, ones_v, acc):
    c = lax.axis_index("c")
    s = lax.axis_index("s")
    wid = _wid()
    pltpu.sync_copy(col_hbm.at[wid], col_v)
    pltpu.sync_copy(ones_hbm, ones_v)

    @pl.when(s < WB_TILES)
    def _init():
        rows = pl.ds(s * WB_ROWS, WB_ROWS)
        pltpu.sync_copy(zeros_hbm, acc.at[rows])

    plsc.subcore_barrier()

    @pl.loop(0, E_CHUNKS)
    def _count(j):
        pltpu.sync_copy(ones_v, acc.at[col_v.at[j]], add=True)

    plsc.subcore_barrier()

    @pl.when(s < WB_TILES)
    def _wb():
        rows = pl.ds(s * WB_ROWS, WB_ROWS)
        pltpu.sync_copy(acc.at[rows], out_hbm.at[c, rows])


_sc_degree = pl.kernel(_sc_degree_body, **_DEG_KW)


def _spmm_kw(C):
    return dict(
        out_type=jax.ShapeDtypeStruct((NC, N, C), jnp.float32),
        mesh=_MESH,
        scratch_types=[
            pltpu.VMEM((E_CHUNKS, CHUNK), jnp.int32),
            pltpu.VMEM((E_CHUNKS, CHUNK), jnp.int32),
            pltpu.VMEM((CHUNK, C), jnp.float32),
            pltpu.VMEM_SHARED((N, C), jnp.float32),
        ],
    )


def _spmm_body(y_hbm, row_hbm, col_hbm, zeros_hbm, out_hbm, row_v, col_v, buf, acc):
        c = lax.axis_index("c")
        s = lax.axis_index("s")
        wid = _wid()
        pltpu.sync_copy(row_hbm.at[wid], row_v)
        pltpu.sync_copy(col_hbm.at[wid], col_v)

        @pl.when(s < WB_TILES)
        def _init():
            rows = pl.ds(s * WB_ROWS, WB_ROWS)
            pltpu.sync_copy(zeros_hbm, acc.at[rows])

        plsc.subcore_barrier()

        @pl.loop(0, E_CHUNKS)
        def _edges(j):
            pltpu.sync_copy(y_hbm.at[row_v.at[j]], buf)
            pltpu.sync_copy(buf, acc.at[col_v.at[j]], add=True)

        plsc.subcore_barrier()

        @pl.when(s < WB_TILES)
        def _wb():
            rows = pl.ds(s * WB_ROWS, WB_ROWS)
            pltpu.sync_copy(acc.at[rows], out_hbm.at[c, rows])

_spmm128 = pl.kernel(_spmm_body, **_spmm_kw(128))


@functools.partial(
    pl.kernel,
    out_type=jax.ShapeDtypeStruct((NW, EL_PER_TILE // 8, 128), jnp.float32),
    mesh=_MESH,
    scratch_types=[
        pltpu.VMEM((EL_CHUNKS, CHUNK_D), jnp.int32),
        pltpu.VMEM((EL_CHUNKS, CHUNK_D), jnp.int32),
        pltpu.VMEM((CHUNK_D, 128), jnp.float32),
        pltpu.VMEM((CHUNK_D, 128), jnp.float32),
        pltpu.VMEM((16, 128), jnp.float32),
    ],
)
def _sc_decode(z_hbm, src_hbm, dst_hbm, out_hbm, src_v, dst_v, abuf, bbuf, tbuf):
    wid = _wid()
    pltpu.sync_copy(src_hbm.at[wid], src_v)
    pltpu.sync_copy(dst_hbm.at[wid], dst_v)

    @pl.loop(0, EL_CHUNKS)
    def _chunk(j):
        pltpu.sync_copy(z_hbm.at[src_v.at[j]], abuf)
        pltpu.sync_copy(z_hbm.at[dst_v.at[j]], bbuf)

        @pl.loop(0, 16, unroll=2)
        def _row(rr):
            for k in range(8):
                r = rr * 8 + k
                tbuf[rr, pl.ds(k * 16, 16)] = (
                    abuf[r, pl.ds(0, 16)] * bbuf[r, pl.ds(0, 16)]
                    + abuf[r, pl.ds(16, 16)] * bbuf[r, pl.ds(16, 16)]
                    + abuf[r, pl.ds(32, 16)] * bbuf[r, pl.ds(32, 16)]
                    + abuf[r, pl.ds(48, 16)] * bbuf[r, pl.ds(48, 16)]
                )

        pltpu.sync_copy(tbuf, out_hbm.at[wid, pl.ds(j * 16, 16)])


def _tc_first_body(cnt_ref, x_ref, w_ref, y_ref, s_ref):
    deg = 1.0 + cnt_ref[0, :, 0:1] + cnt_ref[1, :, 0:1]
    s = lax.rsqrt(deg)
    s_ref[...] = s
    y_ref[...] = s * jnp.dot(x_ref[...], w_ref[...], preferred_element_type=jnp.float32)


def _tc_mid_body(u_ref, y_ref, s_ref, b_ref, w_ref, o_ref):
    s = s_ref[...]
    a = jnp.maximum(s * (u_ref[0] + u_ref[1] + y_ref[...]) + b_ref[...], 0.0)
    o_ref[...] = s * jnp.dot(a, w_ref[...], preferred_element_type=jnp.float32)


def _tc_last_body(u_ref, y_ref, s_ref, b_ref, o_ref):
    o_ref[...] = s_ref[...] * (u_ref[0] + u_ref[1] + y_ref[...]) + b_ref[...]


def _tc_first(cnt, x, W1):
    return pl.pallas_call(
        _tc_first_body,
        out_shape=(
            jax.ShapeDtypeStruct((N, W1.shape[1]), jnp.float32),
            jax.ShapeDtypeStruct((N, 1), jnp.float32),
        ),
    )(cnt, x, W1)


def _tc_mid(u, y, s, b, W):
    return pl.pallas_call(
        _tc_mid_body,
        out_shape=jax.ShapeDtypeStruct((N, W.shape[1]), jnp.float32),
    )(u, y, s, b.reshape(1, -1), W)


def _tc_last(u, y, s, b):
    return pl.pallas_call(
        _tc_last_body,
        out_shape=jax.ShapeDtypeStruct((N, y.shape[1]), jnp.float32),
    )(u, y, s, b.reshape(1, -1))


def _tc_fold_body(t_ref, o_ref):
    m = (lax.broadcasted_iota(jnp.int32, (128, 8), 0) // 16
         == lax.broadcasted_iota(jnp.int32, (128, 8), 1)).astype(jnp.float32)
    o_ref[...] = jnp.dot(t_ref[...], m, precision=lax.Precision.HIGHEST,
                         preferred_element_type=jnp.float32)


def _tc_fold(t):
    Q = NW * EL_PER_TILE // 8
    BQ = Q // 8
    return pl.pallas_call(
        _tc_fold_body,
        grid=(Q // BQ,),
        in_specs=[pl.BlockSpec((BQ, 128), lambda i: (i, 0))],
        out_specs=pl.BlockSpec((BQ, 8), lambda i: (i, 0)),
        out_shape=jax.ShapeDtypeStruct((Q, 8), jnp.float32),
    )(t.reshape(Q, 128))


def kernel(x, edge_index, edge_label_index, W1, b1, W2, b2, W3, b3):
    row3d = edge_index[0].reshape(NW, E_CHUNKS, CHUNK)
    col3d = edge_index[1].reshape(NW, E_CHUNKS, CHUNK)
    eli_pad = jnp.concatenate(
        [edge_label_index, jnp.zeros((2, ELP - EL), jnp.int32)], axis=1)
    src3d = eli_pad[0].reshape(NW, EL_CHUNKS, CHUNK_D)
    dst3d = eli_pad[1].reshape(NW, EL_CHUNKS, CHUNK_D)
    z128 = jnp.zeros((WB_ROWS, 128), jnp.float32)

    ones1 = jnp.ones((CHUNK, DEG_W), jnp.float32)
    z1 = jnp.zeros((WB_ROWS, DEG_W), jnp.float32)
    cnt = _sc_degree(col3d, ones1, z1)
    y1, s = _tc_first(cnt, x, W1)
    u1 = _spmm128(y1, row3d, col3d, z128)
    y2 = _tc_mid(u1, y1, s, b1, W2)
    u2 = _spmm128(y2, row3d, col3d, z128)
    W3p = jnp.concatenate([W3, jnp.zeros((128, 64), jnp.float32)], axis=1)
    b3p = jnp.concatenate([b3, jnp.zeros((64,), jnp.float32)])
    y3 = _tc_mid(u2, y2, s, b2, W3p)
    u3 = _spmm128(y3, row3d, col3d, z128)
    z = _tc_last(u3, y3, s, b3p)
    t = _sc_decode(z, src3d, dst3d)
    out = _tc_fold(t)
    return out.reshape(ELP)[:EL]


# speedup vs baseline: 17.8070x; 1.9075x over previous
import functools

import jax
import jax.numpy as jnp
from jax import lax
from jax.experimental import pallas as pl
from jax.experimental.pallas import tpu as pltpu
from jax.experimental.pallas import tpu_sc as plsc

N = 10000
E = 320000
EL = 200000
NC, NS = 2, 16
NW = NC * NS
CHUNK = 125
WB_TILES = 10
WB_ROWS = N // WB_TILES
E_PER_TILE = E // NW
E_CHUNKS = E_PER_TILE // CHUNK
ELP = 204800
EL_PER_TILE = ELP // NW
CHUNK_D = 128
EL_CHUNKS = EL_PER_TILE // CHUNK_D
DEG_W = 128
NBUF = 2
KDEG = 10

_MESH = plsc.VectorSubcoreMesh(core_axis_name="c", subcore_axis_name="s")


def _wid():
    return lax.axis_index("s") * NC + lax.axis_index("c")


_DEG_KW = dict(
    out_type=jax.ShapeDtypeStruct((NC, N, DEG_W), jnp.float32),
    mesh=_MESH,
    scratch_types=[
        pltpu.VMEM((E_CHUNKS, CHUNK), jnp.int32),
        pltpu.VMEM((CHUNK, DEG_W), jnp.float32),
        pltpu.VMEM_SHARED((N, DEG_W), jnp.float32),
        pltpu.SemaphoreType.DMA,
    ],
)


def _sc_degree_body(col_hbm, ones_hbm, zeros_hbm, out_hbm, col_v, ones_v, acc, sem):
    c = lax.axis_index("c")
    s = lax.axis_index("s")
    wid = _wid()
    pltpu.sync_copy(col_hbm.at[wid], col_v)
    pltpu.sync_copy(ones_hbm, ones_v)

    @pl.when(s < WB_TILES)
    def _init():
        rows = pl.ds(s * WB_ROWS, WB_ROWS)
        pltpu.sync_copy(zeros_hbm, acc.at[rows])

    plsc.subcore_barrier()

    @pl.loop(0, E_CHUNKS // KDEG)
    def _blk(j0):
        descs = []
        for b in range(KDEG):
            j = j0 * KDEG + b
            descs.append(pltpu.async_copy(ones_v, acc.at[col_v.at[j]], sem, add=True))
        for d in descs:
            d.wait()

    plsc.subcore_barrier()

    @pl.when(s < WB_TILES)
    def _wb():
        rows = pl.ds(s * WB_ROWS, WB_ROWS)
        pltpu.sync_copy(acc.at[rows], out_hbm.at[c, rows])


_sc_degree = pl.kernel(_sc_degree_body, **_DEG_KW)


def _spmm_kw(C):
    return dict(
        out_type=jax.ShapeDtypeStruct((NC, N, C), jnp.float32),
        mesh=_MESH,
        scratch_types=[
            pltpu.VMEM((E_CHUNKS // 2, CHUNK), jnp.int32),
            pltpu.VMEM((E_CHUNKS // 2, CHUNK), jnp.int32),
        ]
        + [pltpu.VMEM((CHUNK, C), jnp.float32)] * NBUF
        + [pltpu.SemaphoreType.DMA] * (2 * NBUF)
        + [pltpu.VMEM_SHARED((N, C), jnp.float32)],
    )


def _spmm_body(y_hbm, row_hbm, col_hbm, zeros_hbm, out_hbm, row_v, col_v, *rest):
    bufs = rest[:NBUF]
    gsems = rest[NBUF:2 * NBUF]
    ssems = rest[2 * NBUF:3 * NBUF]
    acc = rest[3 * NBUF]
    c = lax.axis_index("c")
    s = lax.axis_index("s")
    wid = _wid()

    @pl.when(s < WB_TILES)
    def _init():
        rows = pl.ds(s * WB_ROWS, WB_ROWS)
        pltpu.sync_copy(zeros_hbm, acc.at[rows])

    plsc.subcore_barrier()

    H = E_CHUNKS // 2
    BLKS = H // NBUF
    for h in range(2):
        pltpu.sync_copy(row_hbm.at[wid, pl.ds(h * H, H)], row_v)
        pltpu.sync_copy(col_hbm.at[wid, pl.ds(h * H, H)], col_v)
        for b in range(NBUF):
            pltpu.async_copy(y_hbm.at[row_v.at[b]], bufs[b], gsems[b])

        @pl.loop(0, BLKS)
        def _blk(j0):
            base = j0 * NBUF
            descs = []
            for b in range(NBUF):
                j = base + b
                pltpu.make_async_copy(y_hbm.at[row_v.at[j]], bufs[b], gsems[b]).wait()
                descs.append(
                    pltpu.async_copy(bufs[b], acc.at[col_v.at[j]], ssems[b], add=True))
            for b in range(NBUF):
                descs[b].wait()

                @pl.when(j0 < BLKS - 1)
                def _next(b=b, base=base):
                    pltpu.async_copy(
                        y_hbm.at[row_v.at[base + NBUF + b]], bufs[b], gsems[b])

    plsc.subcore_barrier()

    @pl.when(s < WB_TILES)
    def _wb():
        rows = pl.ds(s * WB_ROWS, WB_ROWS)
        pltpu.sync_copy(acc.at[rows], out_hbm.at[c, rows])


_spmm128 = pl.kernel(_spmm_body, **_spmm_kw(128))


_DEC_KW = dict(
    out_type=jax.ShapeDtypeStruct((NW, EL_PER_TILE // 8, 128), jnp.float32),
    mesh=_MESH,
    scratch_types=[
        pltpu.VMEM((EL_CHUNKS, CHUNK_D), jnp.int32),
        pltpu.VMEM((EL_CHUNKS, CHUNK_D), jnp.int32),
        pltpu.VMEM((CHUNK_D, 128), jnp.float32),
        pltpu.VMEM((CHUNK_D, 128), jnp.float32),
        pltpu.VMEM((CHUNK_D, 128), jnp.float32),
        pltpu.VMEM((CHUNK_D, 128), jnp.float32),
        pltpu.VMEM((16, 128), jnp.float32),
        pltpu.VMEM((16, 128), jnp.float32),
        pltpu.SemaphoreType.DMA,
        pltpu.SemaphoreType.DMA,
        pltpu.SemaphoreType.DMA,
        pltpu.SemaphoreType.DMA,
        pltpu.SemaphoreType.DMA,
        pltpu.SemaphoreType.DMA,
    ],
)


def _sc_decode_body(z_hbm, src_hbm, dst_hbm, out_hbm, src_v, dst_v,
                    abuf0, abuf1, bbuf0, bbuf1, tbuf0, tbuf1,
                    ga0, ga1, gb0, gb1, ws0, ws1):
    wid = _wid()
    abufs, bbufs, tbufs = (abuf0, abuf1), (bbuf0, bbuf1), (tbuf0, tbuf1)
    gas, gbs, wss = (ga0, ga1), (gb0, gb1), (ws0, ws1)
    pltpu.sync_copy(src_hbm.at[wid], src_v)
    pltpu.sync_copy(dst_hbm.at[wid], dst_v)
    pltpu.async_copy(z_hbm.at[src_v.at[0]], abufs[0], gas[0])
    pltpu.async_copy(z_hbm.at[dst_v.at[0]], bbufs[0], gbs[0])

    @pl.loop(0, EL_CHUNKS // 2)
    def _blk(j0):
        for b in range(2):
            j = 2 * j0 + b
            pltpu.make_async_copy(z_hbm.at[src_v.at[j]], abufs[b], gas[b]).wait()
            pltpu.make_async_copy(z_hbm.at[dst_v.at[j]], bbufs[b], gbs[b]).wait()

            @pl.when(j < EL_CHUNKS - 1)
            def _nextg(b=b, j=j):
                nb = 1 - b
                pltpu.async_copy(z_hbm.at[src_v.at[j + 1]], abufs[nb], gas[nb])
                pltpu.async_copy(z_hbm.at[dst_v.at[j + 1]], bbufs[nb], gbs[nb])

            @pl.when(j >= 2)
            def _wbwait(b=b, j=j):
                pltpu.make_async_copy(
                    tbufs[b], out_hbm.at[wid, pl.ds((j - 2) * 16, 16)], wss[b]).wait()

            abuf, bbuf, tbuf = abufs[b], bbufs[b], tbufs[b]

            @pl.loop(0, 16, unroll=2)
            def _row(rr):
                for k in range(8):
                    r = rr * 8 + k
                    tbuf[rr, pl.ds(k * 16, 16)] = (
                        abuf[r, pl.ds(0, 16)] * bbuf[r, pl.ds(0, 16)]
                        + abuf[r, pl.ds(16, 16)] * bbuf[r, pl.ds(16, 16)]
                        + abuf[r, pl.ds(32, 16)] * bbuf[r, pl.ds(32, 16)]
                        + abuf[r, pl.ds(48, 16)] * bbuf[r, pl.ds(48, 16)]
                    )

            pltpu.async_copy(tbuf, out_hbm.at[wid, pl.ds(j * 16, 16)], wss[b])

    for j in (EL_CHUNKS - 2, EL_CHUNKS - 1):
        b = j % 2
        pltpu.make_async_copy(
            tbufs[b], out_hbm.at[wid, pl.ds(j * 16, 16)], wss[b]).wait()


_sc_decode = pl.kernel(_sc_decode_body, **_DEC_KW)


def _tc_first_body(cnt_ref, x_ref, w_ref, y_ref, s_ref):
    deg = 1.0 + cnt_ref[0, :, 0:1] + cnt_ref[1, :, 0:1]
    s = lax.rsqrt(deg)
    s_ref[...] = s
    y_ref[...] = s * jnp.dot(x_ref[...], w_ref[...], preferred_element_type=jnp.float32)


def _tc_mid_body(u_ref, y_ref, s_ref, b_ref, w_ref, o_ref):
    s = s_ref[...]
    a = jnp.maximum(s * (u_ref[0] + u_ref[1] + y_ref[...]) + b_ref[...], 0.0)
    o_ref[...] = s * jnp.dot(a, w_ref[...], preferred_element_type=jnp.float32)


def _tc_last_body(u_ref, y_ref, s_ref, b_ref, o_ref):
    o_ref[...] = s_ref[...] * (u_ref[0] + u_ref[1] + y_ref[...]) + b_ref[...]


def _tc_first(cnt, x, W1):
    return pl.pallas_call(
        _tc_first_body,
        out_shape=(
            jax.ShapeDtypeStruct((N, W1.shape[1]), jnp.float32),
            jax.ShapeDtypeStruct((N, 1), jnp.float32),
        ),
    )(cnt, x, W1)


def _tc_mid(u, y, s, b, W):
    return pl.pallas_call(
        _tc_mid_body,
        out_shape=jax.ShapeDtypeStruct((N, W.shape[1]), jnp.float32),
    )(u, y, s, b.reshape(1, -1), W)


def _tc_last(u, y, s, b):
    return pl.pallas_call(
        _tc_last_body,
        out_shape=jax.ShapeDtypeStruct((N, y.shape[1]), jnp.float32),
    )(u, y, s, b.reshape(1, -1))


def _tc_fold_body(t_ref, o_ref):
    m = (lax.broadcasted_iota(jnp.int32, (128, 8), 0) // 16
         == lax.broadcasted_iota(jnp.int32, (128, 8), 1)).astype(jnp.float32)
    o_ref[...] = jnp.dot(t_ref[...], m, precision=lax.Precision.HIGHEST,
                         preferred_element_type=jnp.float32)


def _tc_fold(t):
    Q = NW * EL_PER_TILE // 8
    BQ = Q // 8
    return pl.pallas_call(
        _tc_fold_body,
        grid=(Q // BQ,),
        in_specs=[pl.BlockSpec((BQ, 128), lambda i: (i, 0))],
        out_specs=pl.BlockSpec((BQ, 8), lambda i: (i, 0)),
        out_shape=jax.ShapeDtypeStruct((Q, 8), jnp.float32),
    )(t.reshape(Q, 128))


def kernel(x, edge_index, edge_label_index, W1, b1, W2, b2, W3, b3):
    row3d = edge_index[0].reshape(NW, E_CHUNKS, CHUNK)
    col3d = edge_index[1].reshape(NW, E_CHUNKS, CHUNK)
    pad = (jnp.arange(ELP - EL, dtype=jnp.int32) * 37) % N
    eli_pad = jnp.concatenate(
        [edge_label_index, jnp.stack([pad, pad])], axis=1)
    src3d = eli_pad[0].reshape(NW, EL_CHUNKS, CHUNK_D)
    dst3d = eli_pad[1].reshape(NW, EL_CHUNKS, CHUNK_D)
    z128 = jnp.zeros((WB_ROWS, 128), jnp.float32)

    ones1 = jnp.ones((CHUNK, DEG_W), jnp.float32)
    z1 = jnp.zeros((WB_ROWS, DEG_W), jnp.float32)
    cnt = _sc_degree(col3d, ones1, z1)
    y1, s = _tc_first(cnt, x, W1)
    u1 = _spmm128(y1, row3d, col3d, z128)
    y2 = _tc_mid(u1, y1, s, b1, W2)
    u2 = _spmm128(y2, row3d, col3d, z128)
    W3p = jnp.concatenate([W3, jnp.zeros((128, 64), jnp.float32)], axis=1)
    b3p = jnp.concatenate([b3, jnp.zeros((64,), jnp.float32)])
    y3 = _tc_mid(u2, y2, s, b2, W3p)
    u3 = _spmm128(y3, row3d, col3d, z128)
    z = _tc_last(u3, y3, s, b3p)
    t = _sc_decode(z, src3d, dst3d)
    out = _tc_fold(t)
    return out.reshape(ELP)[:EL]


# speedup vs baseline: 22.5480x; 1.2662x over previous
import functools

import jax
import jax.numpy as jnp
from jax import lax
from jax.experimental import pallas as pl
from jax.experimental.pallas import tpu as pltpu
from jax.experimental.pallas import tpu_sc as plsc

N = 10000
E = 320000
EL = 200000
NC, NS = 2, 16
NW = NC * NS
CHUNK = 125
WB_TILES = 10
WB_ROWS = N // WB_TILES
E_PER_TILE = E // NW
E_CHUNKS = E_PER_TILE // CHUNK
ELP = 204800
EL_PER_TILE = ELP // NW
CHUNK_D = 128
EL_CHUNKS = EL_PER_TILE // CHUNK_D
DEG_W = 16
NBUF = 2
KDEG = 10

_MESH = plsc.VectorSubcoreMesh(core_axis_name="c", subcore_axis_name="s")
_CP = pltpu.CompilerParams(use_tc_tiling_on_sc=False)


def _wid():
    return lax.axis_index("s") * NC + lax.axis_index("c")


_DEG_KW = dict(
    out_type=jax.ShapeDtypeStruct((NC, N, DEG_W), jnp.float32),
    mesh=_MESH,
    compiler_params=_CP,
    scratch_types=[
        pltpu.VMEM((E_CHUNKS, CHUNK), jnp.int32),
        pltpu.VMEM((CHUNK, DEG_W), jnp.float32),
        pltpu.VMEM_SHARED((N, DEG_W), jnp.float32),
        pltpu.SemaphoreType.DMA,
    ],
)


def _sc_degree_body(col_hbm, ones_hbm, zeros_hbm, out_hbm, col_v, ones_v, acc, sem):
    c = lax.axis_index("c")
    s = lax.axis_index("s")
    wid = _wid()
    pltpu.sync_copy(col_hbm.at[wid], col_v)
    pltpu.sync_copy(ones_hbm, ones_v)

    @pl.when(s < WB_TILES)
    def _init():
        rows = pl.ds(s * WB_ROWS, WB_ROWS)
        pltpu.sync_copy(zeros_hbm, acc.at[rows])

    plsc.subcore_barrier()

    @pl.loop(0, E_CHUNKS // KDEG)
    def _blk(j0):
        descs = []
        for b in range(KDEG):
            j = j0 * KDEG + b
            descs.append(pltpu.async_copy(ones_v, acc.at[col_v.at[j]], sem, add=True))
        for d in descs:
            d.wait()

    plsc.subcore_barrier()

    @pl.when(s < WB_TILES)
    def _wb():
        rows = pl.ds(s * WB_ROWS, WB_ROWS)
        pltpu.sync_copy(acc.at[rows], out_hbm.at[c, rows])


_sc_degree = pl.kernel(_sc_degree_body, **_DEG_KW)


def _spmm_kw(C, nbuf):
    return dict(
        out_type=jax.ShapeDtypeStruct((NC, N, C), jnp.float32),
        mesh=_MESH,
        compiler_params=_CP,
        scratch_types=[
            pltpu.VMEM((E_CHUNKS // 2, CHUNK), jnp.int32),
            pltpu.VMEM((E_CHUNKS // 2, CHUNK), jnp.int32),
        ]
        + [pltpu.VMEM((CHUNK, C), jnp.float32)] * nbuf
        + [pltpu.SemaphoreType.DMA] * (2 * nbuf)
        + [pltpu.VMEM_SHARED((N, C), jnp.float32)],
    )


def _make_spmm_body(nbuf):
  def _spmm_body(y_hbm, row_hbm, col_hbm, zeros_hbm, out_hbm, row_v, col_v, *rest):
    bufs = rest[:nbuf]
    gsems = rest[nbuf:2 * nbuf]
    ssems = rest[2 * nbuf:3 * nbuf]
    acc = rest[3 * nbuf]
    c = lax.axis_index("c")
    s = lax.axis_index("s")
    wid = _wid()

    @pl.when(s < WB_TILES)
    def _init():
        rows = pl.ds(s * WB_ROWS, WB_ROWS)
        pltpu.sync_copy(zeros_hbm, acc.at[rows])

    plsc.subcore_barrier()

    H = E_CHUNKS // 2
    BLKS = H // nbuf
    for h in range(2):
        pltpu.sync_copy(row_hbm.at[wid, pl.ds(h * H, H)], row_v)
        pltpu.sync_copy(col_hbm.at[wid, pl.ds(h * H, H)], col_v)
        for b in range(nbuf):
            pltpu.async_copy(y_hbm.at[row_v.at[b]], bufs[b], gsems[b])

        @pl.loop(0, BLKS)
        def _blk(j0):
            base = j0 * nbuf
            descs = []
            for b in range(nbuf):
                j = base + b
                pltpu.make_async_copy(y_hbm.at[row_v.at[j]], bufs[b], gsems[b]).wait()
                descs.append(
                    pltpu.async_copy(bufs[b], acc.at[col_v.at[j]], ssems[b], add=True))
            for b in range(nbuf):
                descs[b].wait()

                @pl.when(j0 < BLKS - 1)
                def _next(b=b, base=base):
                    pltpu.async_copy(
                        y_hbm.at[row_v.at[base + nbuf + b]], bufs[b], gsems[b])

    plsc.subcore_barrier()

    @pl.when(s < WB_TILES)
    def _wb():
        rows = pl.ds(s * WB_ROWS, WB_ROWS)
        pltpu.sync_copy(acc.at[rows], out_hbm.at[c, rows])

  return _spmm_body


_spmm128 = pl.kernel(_make_spmm_body(2), **_spmm_kw(128, 2))
_spmm64 = pl.kernel(_make_spmm_body(4), **_spmm_kw(64, 4))


_DEC_KW = dict(
    out_type=jax.ShapeDtypeStruct((NW, EL_PER_TILE // 8, 128), jnp.float32),
    mesh=_MESH,
    compiler_params=_CP,
    scratch_types=[
        pltpu.VMEM((EL_CHUNKS, CHUNK_D), jnp.int32),
        pltpu.VMEM((EL_CHUNKS, CHUNK_D), jnp.int32),
        pltpu.VMEM((CHUNK_D, 64), jnp.float32),
        pltpu.VMEM((CHUNK_D, 64), jnp.float32),
        pltpu.VMEM((CHUNK_D, 64), jnp.float32),
        pltpu.VMEM((CHUNK_D, 64), jnp.float32),
        pltpu.VMEM((16, 128), jnp.float32),
        pltpu.VMEM((16, 128), jnp.float32),
        pltpu.SemaphoreType.DMA,
        pltpu.SemaphoreType.DMA,
        pltpu.SemaphoreType.DMA,
        pltpu.SemaphoreType.DMA,
        pltpu.SemaphoreType.DMA,
        pltpu.SemaphoreType.DMA,
    ],
)


def _sc_decode_body(z_hbm, src_hbm, dst_hbm, out_hbm, src_v, dst_v,
                    abuf0, abuf1, bbuf0, bbuf1, tbuf0, tbuf1,
                    ga0, ga1, gb0, gb1, ws0, ws1):
    wid = _wid()
    abufs, bbufs, tbufs = (abuf0, abuf1), (bbuf0, bbuf1), (tbuf0, tbuf1)
    gas, gbs, wss = (ga0, ga1), (gb0, gb1), (ws0, ws1)
    pltpu.sync_copy(src_hbm.at[wid], src_v)
    pltpu.sync_copy(dst_hbm.at[wid], dst_v)
    pltpu.async_copy(z_hbm.at[src_v.at[0]], abufs[0], gas[0])
    pltpu.async_copy(z_hbm.at[dst_v.at[0]], bbufs[0], gbs[0])

    @pl.loop(0, EL_CHUNKS // 2)
    def _blk(j0):
        for b in range(2):
            j = 2 * j0 + b
            pltpu.make_async_copy(z_hbm.at[src_v.at[j]], abufs[b], gas[b]).wait()
            pltpu.make_async_copy(z_hbm.at[dst_v.at[j]], bbufs[b], gbs[b]).wait()

            @pl.when(j < EL_CHUNKS - 1)
            def _nextg(b=b, j=j):
                nb = 1 - b
                pltpu.async_copy(z_hbm.at[src_v.at[j + 1]], abufs[nb], gas[nb])
                pltpu.async_copy(z_hbm.at[dst_v.at[j + 1]], bbufs[nb], gbs[nb])

            @pl.when(j >= 2)
            def _wbwait(b=b, j=j):
                pltpu.make_async_copy(
                    tbufs[b], out_hbm.at[wid, pl.ds((j - 2) * 16, 16)], wss[b]).wait()

            abuf, bbuf, tbuf = abufs[b], bbufs[b], tbufs[b]

            @pl.loop(0, 16, unroll=2)
            def _row(rr):
                for k in range(8):
                    r = rr * 8 + k
                    tbuf[rr, pl.ds(k * 16, 16)] = (
                        abuf[r, pl.ds(0, 16)] * bbuf[r, pl.ds(0, 16)]
                        + abuf[r, pl.ds(16, 16)] * bbuf[r, pl.ds(16, 16)]
                        + abuf[r, pl.ds(32, 16)] * bbuf[r, pl.ds(32, 16)]
                        + abuf[r, pl.ds(48, 16)] * bbuf[r, pl.ds(48, 16)]
                    )

            pltpu.async_copy(tbuf, out_hbm.at[wid, pl.ds(j * 16, 16)], wss[b])

    for j in (EL_CHUNKS - 2, EL_CHUNKS - 1):
        b = j % 2
        pltpu.make_async_copy(
            tbufs[b], out_hbm.at[wid, pl.ds(j * 16, 16)], wss[b]).wait()


_sc_decode = pl.kernel(_sc_decode_body, **_DEC_KW)


def _tc_first_body(cnt_ref, x_ref, w_ref, y_ref, s_ref):
    deg = 1.0 + cnt_ref[0, :, 0:1] + cnt_ref[1, :, 0:1]
    s = lax.rsqrt(deg)
    s_ref[...] = s
    y_ref[...] = s * jnp.dot(x_ref[...], w_ref[...], preferred_element_type=jnp.float32)


def _tc_mid_body(u_ref, y_ref, s_ref, b_ref, w_ref, o_ref):
    s = s_ref[...]
    a = jnp.maximum(s * (u_ref[0] + u_ref[1] + y_ref[...]) + b_ref[...], 0.0)
    o_ref[...] = s * jnp.dot(a, w_ref[...], preferred_element_type=jnp.float32)


def _tc_last_body(u_ref, y_ref, s_ref, b_ref, o_ref):
    o_ref[...] = s_ref[...] * (u_ref[0] + u_ref[1] + y_ref[...]) + b_ref[...]


def _tc_first(cnt, x, W1):
    return pl.pallas_call(
        _tc_first_body,
        out_shape=(
            jax.ShapeDtypeStruct((N, W1.shape[1]), jnp.float32),
            jax.ShapeDtypeStruct((N, 1), jnp.float32),
        ),
    )(cnt, x, W1)


def _tc_mid(u, y, s, b, W):
    return pl.pallas_call(
        _tc_mid_body,
        out_shape=jax.ShapeDtypeStruct((N, W.shape[1]), jnp.float32),
    )(u, y, s, b.reshape(1, -1), W)


def _tc_last(u, y, s, b):
    return pl.pallas_call(
        _tc_last_body,
        out_shape=jax.ShapeDtypeStruct((N, y.shape[1]), jnp.float32),
    )(u, y, s, b.reshape(1, -1))


def _tc_fold_body(t_ref, o_ref):
    m = (lax.broadcasted_iota(jnp.int32, (128, 8), 0) // 16
         == lax.broadcasted_iota(jnp.int32, (128, 8), 1)).astype(jnp.float32)
    o_ref[...] = jnp.dot(t_ref[...], m, precision=lax.Precision.HIGHEST,
                         preferred_element_type=jnp.float32)


def _tc_fold(t):
    Q = NW * EL_PER_TILE // 8
    BQ = Q // 8
    return pl.pallas_call(
        _tc_fold_body,
        grid=(Q // BQ,),
        in_specs=[pl.BlockSpec((BQ, 128), lambda i: (i, 0))],
        out_specs=pl.BlockSpec((BQ, 8), lambda i: (i, 0)),
        out_shape=jax.ShapeDtypeStruct((Q, 8), jnp.float32),
    )(t.reshape(Q, 128))


def kernel(x, edge_index, edge_label_index, W1, b1, W2, b2, W3, b3):
    row3d = edge_index[0].reshape(NW, E_CHUNKS, CHUNK)
    col3d = edge_index[1].reshape(NW, E_CHUNKS, CHUNK)
    pad = (jnp.arange(ELP - EL, dtype=jnp.int32) * 37) % N
    eli_pad = jnp.concatenate(
        [edge_label_index, jnp.stack([pad, pad])], axis=1)
    src3d = eli_pad[0].reshape(NW, EL_CHUNKS, CHUNK_D)
    dst3d = eli_pad[1].reshape(NW, EL_CHUNKS, CHUNK_D)
    z128 = jnp.zeros((WB_ROWS, 128), jnp.float32)

    ones1 = jnp.ones((CHUNK, DEG_W), jnp.float32)
    z1 = jnp.zeros((WB_ROWS, DEG_W), jnp.float32)
    cnt = _sc_degree(col3d, ones1, z1)
    y1, s = _tc_first(cnt, x, W1)
    u1 = _spmm128(y1, row3d, col3d, z128)
    y2 = _tc_mid(u1, y1, s, b1, W2)
    u2 = _spmm128(y2, row3d, col3d, z128)
    y3 = _tc_mid(u2, y2, s, b2, W3)
    z64 = jnp.zeros((WB_ROWS, 64), jnp.float32)
    u3 = _spmm64(y3, row3d, col3d, z64)
    z = _tc_last(u3, y3, s, b3)
    t = _sc_decode(z, src3d, dst3d)
    out = _tc_fold(t)
    return out.reshape(ELP)[:EL]


# speedup vs baseline: 23.5532x; 1.0446x over previous
import functools

import jax
import jax.numpy as jnp
from jax import lax
from jax.experimental import pallas as pl
from jax.experimental.pallas import tpu as pltpu
from jax.experimental.pallas import tpu_sc as plsc

N = 10000
E = 320000
EL = 200000
NC, NS = 2, 16
NW = NC * NS
CHUNK = 125
WB_TILES = 10
WB_ROWS = N // WB_TILES
E_PER_TILE = E // NW
E_CHUNKS = E_PER_TILE // CHUNK
ELP = 204800
EL_PER_TILE = ELP // NW
CHUNK_D = 128
EL_CHUNKS = EL_PER_TILE // CHUNK_D
DEG_W = 16
NBUF = 2
CS_CHUNKS = 160
HQ = 40
NB_CS = 8
KDEG = 10

_MESH = plsc.VectorSubcoreMesh(core_axis_name="c", subcore_axis_name="s")
_CP = pltpu.CompilerParams(use_tc_tiling_on_sc=False)


def _wid():
    return lax.axis_index("s") * NC + lax.axis_index("c")


_DEG_KW = dict(
    out_type=jax.ShapeDtypeStruct((NC, N, DEG_W), jnp.float32),
    mesh=_MESH,
    compiler_params=_CP,
    scratch_types=[
        pltpu.VMEM((E_CHUNKS, CHUNK), jnp.int32),
        pltpu.VMEM((CHUNK, DEG_W), jnp.float32),
        pltpu.VMEM_SHARED((N, DEG_W), jnp.float32),
        pltpu.SemaphoreType.DMA,
    ],
)


def _sc_degree_body(col_hbm, ones_hbm, zeros_hbm, out_hbm, col_v, ones_v, acc, sem):
    c = lax.axis_index("c")
    s = lax.axis_index("s")
    wid = _wid()
    pltpu.sync_copy(col_hbm.at[wid], col_v)
    pltpu.sync_copy(ones_hbm, ones_v)

    @pl.when(s < WB_TILES)
    def _init():
        rows = pl.ds(s * WB_ROWS, WB_ROWS)
        pltpu.sync_copy(zeros_hbm, acc.at[rows])

    plsc.subcore_barrier()

    @pl.loop(0, E_CHUNKS // KDEG)
    def _blk(j0):
        descs = []
        for b in range(KDEG):
            j = j0 * KDEG + b
            descs.append(pltpu.async_copy(ones_v, acc.at[col_v.at[j]], sem, add=True))
        for d in descs:
            d.wait()

    plsc.subcore_barrier()

    @pl.when(s < WB_TILES)
    def _wb():
        rows = pl.ds(s * WB_ROWS, WB_ROWS)
        pltpu.sync_copy(acc.at[rows], out_hbm.at[c, rows])


_sc_degree = pl.kernel(_sc_degree_body, **_DEG_KW)


def _spmm_kw(C, nbuf):
    return dict(
        out_type=jax.ShapeDtypeStruct((NC, N, C), jnp.float32),
        mesh=_MESH,
        compiler_params=_CP,
        scratch_types=[
            pltpu.VMEM((E_CHUNKS // 2, CHUNK), jnp.int32),
            pltpu.VMEM((E_CHUNKS // 2, CHUNK), jnp.int32),
        ]
        + [pltpu.VMEM((CHUNK, C), jnp.float32)] * nbuf
        + [pltpu.SemaphoreType.DMA] * (2 * nbuf)
        + [pltpu.VMEM_SHARED((N, C), jnp.float32)],
    )


def _make_spmm_body(nbuf):
  def _spmm_body(y_hbm, row_hbm, col_hbm, zeros_hbm, out_hbm, row_v, col_v, *rest):
    bufs = rest[:nbuf]
    gsems = rest[nbuf:2 * nbuf]
    ssems = rest[2 * nbuf:3 * nbuf]
    acc = rest[3 * nbuf]
    c = lax.axis_index("c")
    s = lax.axis_index("s")
    wid = _wid()

    @pl.when(s < WB_TILES)
    def _init():
        rows = pl.ds(s * WB_ROWS, WB_ROWS)
        pltpu.sync_copy(zeros_hbm, acc.at[rows])

    plsc.subcore_barrier()

    H = E_CHUNKS // 2
    BLKS = H // nbuf
    for h in range(2):
        pltpu.sync_copy(row_hbm.at[wid, pl.ds(h * H, H)], row_v)
        pltpu.sync_copy(col_hbm.at[wid, pl.ds(h * H, H)], col_v)
        for b in range(nbuf):
            pltpu.async_copy(y_hbm.at[row_v.at[b]], bufs[b], gsems[b])

        @pl.loop(0, BLKS)
        def _blk(j0):
            base = j0 * nbuf
            descs = []
            for b in range(nbuf):
                j = base + b
                pltpu.make_async_copy(y_hbm.at[row_v.at[j]], bufs[b], gsems[b]).wait()
                descs.append(
                    pltpu.async_copy(bufs[b], acc.at[col_v.at[j]], ssems[b], add=True))
            for b in range(nbuf):
                descs[b].wait()

                @pl.when(j0 < BLKS - 1)
                def _next(b=b, base=base):
                    pltpu.async_copy(
                        y_hbm.at[row_v.at[base + nbuf + b]], bufs[b], gsems[b])

    plsc.subcore_barrier()

    @pl.when(s < WB_TILES)
    def _wb():
        rows = pl.ds(s * WB_ROWS, WB_ROWS)
        pltpu.sync_copy(acc.at[rows], out_hbm.at[c, rows])

  return _spmm_body


_spmm128 = pl.kernel(_make_spmm_body(2), **_spmm_kw(128, 2))
_spmm64 = pl.kernel(_make_spmm_body(4), **_spmm_kw(64, 4))


_CS_KW = dict(
    out_type=jax.ShapeDtypeStruct((NC, N, 64), jnp.float32),
    mesh=_MESH,
    compiler_params=_CP,
    scratch_types=[
        pltpu.VMEM((HQ, CHUNK), jnp.int32),
        pltpu.VMEM((HQ, CHUNK), jnp.int32),
    ]
    + [pltpu.VMEM((CHUNK, 64), jnp.float32)] * NB_CS
    + [pltpu.SemaphoreType.DMA] * (2 * NB_CS)
    + [pltpu.VMEM_SHARED((N, 64), jnp.float32)],
)


def _cs_body(ylo_hbm, yhi_hbm, row_hbm, col_hbm, zeros_hbm, out_hbm,
             row_v, col_v, *rest):
    bufs = rest[:NB_CS]
    gsems = rest[NB_CS:2 * NB_CS]
    ssems = rest[2 * NB_CS:3 * NB_CS]
    acc = rest[3 * NB_CS]
    c = lax.axis_index("c")
    s = lax.axis_index("s")

    @pl.when(s < WB_TILES)
    def _init():
        rows = pl.ds(s * WB_ROWS, WB_ROWS)
        pltpu.sync_copy(zeros_hbm, acc.at[rows])

    plsc.subcore_barrier()

    BLKS = HQ // NB_CS

    def _run(y_hbm):
        for h in range(CS_CHUNKS // HQ):
            pltpu.sync_copy(row_hbm.at[s, pl.ds(h * HQ, HQ)], row_v)
            pltpu.sync_copy(col_hbm.at[s, pl.ds(h * HQ, HQ)], col_v)
            for b in range(NB_CS):
                pltpu.async_copy(y_hbm.at[row_v.at[b]], bufs[b], gsems[b])

            @pl.loop(0, BLKS)
            def _blk(j0):
                base = j0 * NB_CS
                descs = []
                for b in range(NB_CS):
                    j = base + b
                    pltpu.make_async_copy(
                        y_hbm.at[row_v.at[j]], bufs[b], gsems[b]).wait()
                    descs.append(pltpu.async_copy(
                        bufs[b], acc.at[col_v.at[j]], ssems[b], add=True))
                for b in range(NB_CS):
                    descs[b].wait()

                    @pl.when(j0 < BLKS - 1)
                    def _next(b=b, base=base):
                        pltpu.async_copy(
                            y_hbm.at[row_v.at[base + NB_CS + b]], bufs[b], gsems[b])

    @pl.when(c == 0)
    def _run0():
        _run(ylo_hbm)

    @pl.when(c == 1)
    def _run1():
        _run(yhi_hbm)

    plsc.subcore_barrier()

    @pl.when(s < WB_TILES)
    def _wb():
        rows = pl.ds(s * WB_ROWS, WB_ROWS)
        pltpu.sync_copy(acc.at[rows], out_hbm.at[c, rows])


_cs_spmm = pl.kernel(_cs_body, **_CS_KW)


_DEC_KW = dict(
    out_type=jax.ShapeDtypeStruct((NW, EL_PER_TILE // 8, 128), jnp.float32),
    mesh=_MESH,
    compiler_params=_CP,
    scratch_types=[
        pltpu.VMEM((EL_CHUNKS, CHUNK_D), jnp.int32),
        pltpu.VMEM((EL_CHUNKS, CHUNK_D), jnp.int32),
        pltpu.VMEM((CHUNK_D, 64), jnp.float32),
        pltpu.VMEM((CHUNK_D, 64), jnp.float32),
        pltpu.VMEM((CHUNK_D, 64), jnp.float32),
        pltpu.VMEM((CHUNK_D, 64), jnp.float32),
        pltpu.VMEM((16, 128), jnp.float32),
        pltpu.VMEM((16, 128), jnp.float32),
        pltpu.SemaphoreType.DMA,
        pltpu.SemaphoreType.DMA,
        pltpu.SemaphoreType.DMA,
        pltpu.SemaphoreType.DMA,
        pltpu.SemaphoreType.DMA,
        pltpu.SemaphoreType.DMA,
    ],
)


def _sc_decode_body(z_hbm, src_hbm, dst_hbm, out_hbm, src_v, dst_v,
                    abuf0, abuf1, bbuf0, bbuf1, tbuf0, tbuf1,
                    ga0, ga1, gb0, gb1, ws0, ws1):
    wid = _wid()
    abufs, bbufs, tbufs = (abuf0, abuf1), (bbuf0, bbuf1), (tbuf0, tbuf1)
    gas, gbs, wss = (ga0, ga1), (gb0, gb1), (ws0, ws1)
    pltpu.sync_copy(src_hbm.at[wid], src_v)
    pltpu.sync_copy(dst_hbm.at[wid], dst_v)
    pltpu.async_copy(z_hbm.at[src_v.at[0]], abufs[0], gas[0])
    pltpu.async_copy(z_hbm.at[dst_v.at[0]], bbufs[0], gbs[0])

    @pl.loop(0, EL_CHUNKS // 2)
    def _blk(j0):
        for b in range(2):
            j = 2 * j0 + b
            pltpu.make_async_copy(z_hbm.at[src_v.at[j]], abufs[b], gas[b]).wait()
            pltpu.make_async_copy(z_hbm.at[dst_v.at[j]], bbufs[b], gbs[b]).wait()

            @pl.when(j < EL_CHUNKS - 1)
            def _nextg(b=b, j=j):
                nb = 1 - b
                pltpu.async_copy(z_hbm.at[src_v.at[j + 1]], abufs[nb], gas[nb])
                pltpu.async_copy(z_hbm.at[dst_v.at[j + 1]], bbufs[nb], gbs[nb])

            @pl.when(j >= 2)
            def _wbwait(b=b, j=j):
                pltpu.make_async_copy(
                    tbufs[b], out_hbm.at[wid, pl.ds((j - 2) * 16, 16)], wss[b]).wait()

            abuf, bbuf, tbuf = abufs[b], bbufs[b], tbufs[b]

            @pl.loop(0, 16, unroll=2)
            def _row(rr):
                for k in range(8):
                    r = rr * 8 + k
                    tbuf[rr, pl.ds(k * 16, 16)] = (
                        abuf[r, pl.ds(0, 16)] * bbuf[r, pl.ds(0, 16)]
                        + abuf[r, pl.ds(16, 16)] * bbuf[r, pl.ds(16, 16)]
                        + abuf[r, pl.ds(32, 16)] * bbuf[r, pl.ds(32, 16)]
                        + abuf[r, pl.ds(48, 16)] * bbuf[r, pl.ds(48, 16)]
                    )

            pltpu.async_copy(tbuf, out_hbm.at[wid, pl.ds(j * 16, 16)], wss[b])

    for j in (EL_CHUNKS - 2, EL_CHUNKS - 1):
        b = j % 2
        pltpu.make_async_copy(
            tbufs[b], out_hbm.at[wid, pl.ds(j * 16, 16)], wss[b]).wait()


_sc_decode = pl.kernel(_sc_decode_body, **_DEC_KW)


def _tc_first_body(cnt_ref, x_ref, w_ref, ylo_ref, yhi_ref, s_ref):
    deg = 1.0 + cnt_ref[0, :, 0:1] + cnt_ref[1, :, 0:1]
    s = lax.rsqrt(deg)
    s_ref[...] = s
    res = s * jnp.dot(x_ref[...], w_ref[...], preferred_element_type=jnp.float32)
    ylo_ref[...] = res[:, :64]
    yhi_ref[...] = res[:, 64:]


def _tc_mid2_body(u_ref, ylo_ref, yhi_ref, s_ref, b_ref, w_ref, olo_ref, ohi_ref):
    s = s_ref[...]
    alo = jnp.maximum(s * (u_ref[0] + ylo_ref[...]) + b_ref[:, :64], 0.0)
    ahi = jnp.maximum(s * (u_ref[1] + yhi_ref[...]) + b_ref[:, 64:], 0.0)
    res = s * (jnp.dot(alo, w_ref[:64, :], preferred_element_type=jnp.float32)
               + jnp.dot(ahi, w_ref[64:, :], preferred_element_type=jnp.float32))
    olo_ref[...] = res[:, :64]
    ohi_ref[...] = res[:, 64:]


def _tc_mid3_body(u_ref, ylo_ref, yhi_ref, s_ref, b_ref, w_ref, o_ref):
    s = s_ref[...]
    alo = jnp.maximum(s * (u_ref[0] + ylo_ref[...]) + b_ref[:, :64], 0.0)
    ahi = jnp.maximum(s * (u_ref[1] + yhi_ref[...]) + b_ref[:, 64:], 0.0)
    o_ref[...] = s * (jnp.dot(alo, w_ref[:64, :], preferred_element_type=jnp.float32)
                      + jnp.dot(ahi, w_ref[64:, :], preferred_element_type=jnp.float32))


def _tc_last_body(u_ref, y_ref, s_ref, b_ref, o_ref):
    o_ref[...] = s_ref[...] * (u_ref[0] + u_ref[1] + y_ref[...]) + b_ref[...]


def _tc_first(cnt, x, W1):
    return pl.pallas_call(
        _tc_first_body,
        out_shape=(
            jax.ShapeDtypeStruct((N, 64), jnp.float32),
            jax.ShapeDtypeStruct((N, 64), jnp.float32),
            jax.ShapeDtypeStruct((N, 1), jnp.float32),
        ),
    )(cnt, x, W1)


def _tc_mid2(u, ylo, yhi, s, b, W):
    return pl.pallas_call(
        _tc_mid2_body,
        out_shape=(
            jax.ShapeDtypeStruct((N, 64), jnp.float32),
            jax.ShapeDtypeStruct((N, 64), jnp.float32),
        ),
    )(u, ylo, yhi, s, b.reshape(1, -1), W)


def _tc_mid3(u, ylo, yhi, s, b, W):
    return pl.pallas_call(
        _tc_mid3_body,
        out_shape=jax.ShapeDtypeStruct((N, W.shape[1]), jnp.float32),
    )(u, ylo, yhi, s, b.reshape(1, -1), W)


def _tc_last(u, y, s, b):
    return pl.pallas_call(
        _tc_last_body,
        out_shape=jax.ShapeDtypeStruct((N, y.shape[1]), jnp.float32),
    )(u, y, s, b.reshape(1, -1))


def _tc_fold_body(t_ref, o_ref):
    m = (lax.broadcasted_iota(jnp.int32, (128, 8), 0) // 16
         == lax.broadcasted_iota(jnp.int32, (128, 8), 1)).astype(jnp.float32)
    o_ref[...] = jnp.dot(t_ref[...], m, precision=lax.Precision.HIGHEST,
                         preferred_element_type=jnp.float32)


def _tc_fold(t):
    Q = NW * EL_PER_TILE // 8
    BQ = Q // 8
    return pl.pallas_call(
        _tc_fold_body,
        grid=(Q // BQ,),
        in_specs=[pl.BlockSpec((BQ, 128), lambda i: (i, 0))],
        out_specs=pl.BlockSpec((BQ, 8), lambda i: (i, 0)),
        out_shape=jax.ShapeDtypeStruct((Q, 8), jnp.float32),
    )(t.reshape(Q, 128))


def kernel(x, edge_index, edge_label_index, W1, b1, W2, b2, W3, b3):
    row3d = edge_index[0].reshape(NW, E_CHUNKS, CHUNK)
    col3d = edge_index[1].reshape(NW, E_CHUNKS, CHUNK)
    rowcs = edge_index[0].reshape(NS, CS_CHUNKS, CHUNK)
    colcs = edge_index[1].reshape(NS, CS_CHUNKS, CHUNK)
    pad = (jnp.arange(ELP - EL, dtype=jnp.int32) * 37) % N
    eli_pad = jnp.concatenate(
        [edge_label_index, jnp.stack([pad, pad])], axis=1)
    src3d = eli_pad[0].reshape(NW, EL_CHUNKS, CHUNK_D)
    dst3d = eli_pad[1].reshape(NW, EL_CHUNKS, CHUNK_D)

    ones1 = jnp.ones((CHUNK, DEG_W), jnp.float32)
    z1 = jnp.zeros((WB_ROWS, DEG_W), jnp.float32)
    z64 = jnp.zeros((WB_ROWS, 64), jnp.float32)
    cnt = _sc_degree(col3d, ones1, z1)
    y1lo, y1hi, s = _tc_first(cnt, x, W1)
    u1 = _cs_spmm(y1lo, y1hi, rowcs, colcs, z64)
    y2lo, y2hi = _tc_mid2(u1, y1lo, y1hi, s, b1, W2)
    u2 = _cs_spmm(y2lo, y2hi, rowcs, colcs, z64)
    y3 = _tc_mid3(u2, y2lo, y2hi, s, b2, W3)
    u3 = _spmm64(y3, row3d, col3d, z64)
    z = _tc_last(u3, y3, s, b3)
    t = _sc_decode(z, src3d, dst3d)
    out = _tc_fold(t)
    return out.reshape(ELP)[:EL]
